# Initial kernel scaffold; baseline (speedup 1.0000x reference)
import jax
import jax.numpy as jnp
from jax.experimental import pallas as pl


def kernel(nodes, edge_index, distance, graph_ids, emb, W1s, Wc1s, bc1s, Wc2s, bc2s, W2s, b2s, W3s, b3s, Wd1, bd1, Wd2, bd2):
    raise NotImplementedError("write your pallas kernel here")


# speedup vs baseline: 3.1776x; 3.1776x over previous
import functools

import jax
import jax.numpy as jnp
from jax import lax
from jax.experimental import pallas as pl
from jax.experimental.pallas import tpu as pltpu
import jax.experimental.pallas.tpu_sc as plsc

N = 10000
E = 320000
NG = 64
DIM = 128
TYPE_NUM = 100
RBF_DIM = 50
NCONV = 3

_NC = 2
_NS = 16
_NW = _NC * _NS
_EPT = E // _NW
_CH = 80
_NCHUNK = _EPT // _CH
_NPAD = 10240
_RPT = _NPAD // _NS
_ZROWS = 128

_LOG2 = 0.6931471805599453


def _sp05(x):
    return jnp.where(0.5 * x > 14.0, x,
                     2.0 * jnp.log1p(jnp.exp(jnp.minimum(0.5 * x, 14.0))))


def _sp1(x):
    return jnp.where(x > 20.0, x, jnp.log1p(jnp.exp(jnp.minimum(x, 20.0))))


def _embed_body(nodes_ref, emb_ref, w1_ref, node_ref, new_ref):
    nv = nodes_ref[0, 0, :]
    oh = (nv[:, None] == lax.broadcasted_iota(jnp.int32, (1000, TYPE_NUM), 1)
          ).astype(jnp.float32)
    node = jnp.dot(oh, emb_ref[...], preferred_element_type=jnp.float32)
    node_ref[...] = node
    new_ref[...] = jnp.dot(node, w1_ref[...], preferred_element_type=jnp.float32)


def _embed_call(nodes, emb, w1):
    nodes_r = nodes.reshape(10, 1, 1000)
    return pl.pallas_call(
        _embed_body,
        grid=(10,),
        in_specs=[
            pl.BlockSpec((1, 1, 1000), lambda i: (i, 0, 0)),
            pl.BlockSpec((TYPE_NUM, DIM), lambda i: (0, 0)),
            pl.BlockSpec((DIM, DIM), lambda i: (0, 0)),
        ],
        out_specs=[
            pl.BlockSpec((1000, DIM), lambda i: (i, 0)),
            pl.BlockSpec((1000, DIM), lambda i: (i, 0)),
        ],
        out_shape=[
            jax.ShapeDtypeStruct((N, DIM), jnp.float32),
            jax.ShapeDtypeStruct((N, DIM), jnp.float32),
        ],
    )(nodes_r, emb, w1)


_ECH = 2000
_ESTEPS = E // _ECH


def _h_body(dist_ref, wc1_ref, bc1_ref, wc2_ref, bc2_ref, h_ref):
    d = dist_ref[0, 0, :]
    gap = 5.0 / (RBF_DIM - 1)
    k = lax.broadcasted_iota(jnp.int32, (_ECH, RBF_DIM), 1).astype(jnp.float32)
    diff = d[:, None] - k * gap
    rbf = jnp.exp((-1.0 / gap) * diff * diff)
    t = _sp05(jnp.dot(rbf, wc1_ref[...], preferred_element_type=jnp.float32)
              + bc1_ref[...])
    h_ref[...] = (jnp.dot(t, wc2_ref[...], preferred_element_type=jnp.float32)
                  + bc2_ref[...])


def _h_call(distance, wc1, bc1, wc2, bc2):
    dist_r = distance.reshape(_ESTEPS, 1, _ECH)
    return pl.pallas_call(
        _h_body,
        grid=(_ESTEPS,),
        in_specs=[
            pl.BlockSpec((1, 1, _ECH), lambda e: (e, 0, 0)),
            pl.BlockSpec((RBF_DIM, DIM), lambda e: (0, 0)),
            pl.BlockSpec((1, DIM), lambda e: (0, 0)),
            pl.BlockSpec((DIM, DIM), lambda e: (0, 0)),
            pl.BlockSpec((1, DIM), lambda e: (0, 0)),
        ],
        out_specs=pl.BlockSpec((_ECH, DIM), lambda e: (e, 0)),
        out_shape=jax.ShapeDtypeStruct((E, DIM), jnp.float32),
    )(dist_r, wc1, bc1.reshape(1, DIM), wc2, bc2.reshape(1, DIM))


def _sc_body(new_hbm, h_hbm, src_hbm, dst_hbm, out_hbm,
             acc, srcv, dstv, gbuf, hbuf, zbuf, sem):
    c = lax.axis_index("c")
    s = lax.axis_index("s")
    wid = c * _NS + s

    def zrow(r, carry):
        for j in range(8):
            zbuf[r, pl.ds(16 * j, 16)] = jnp.zeros((16,), jnp.float32)
        return carry
    lax.fori_loop(0, _ZROWS, zrow, 0)
    for kk in range(_RPT // _ZROWS):
        pltpu.sync_copy(zbuf, acc.at[pl.ds(s * _RPT + kk * _ZROWS, _ZROWS)])
    plsc.subcore_barrier()

    def chunk(g, carry):
        base = wid * _EPT + g * _CH
        pltpu.sync_copy(src_hbm.at[pl.ds(base, _CH)], srcv)
        pltpu.sync_copy(dst_hbm.at[pl.ds(base, _CH)], dstv)
        cp = pltpu.async_copy(new_hbm.at[srcv], gbuf, sem)
        pltpu.sync_copy(h_hbm.at[pl.ds(base, _CH)], hbuf)
        cp.wait()

        def mulrow(r, cc):
            for j in range(8):
                sl = pl.ds(16 * j, 16)
                gbuf[r, sl] = gbuf[r, sl] * hbuf[r, sl]
            return cc
        lax.fori_loop(0, _CH, mulrow, 0)
        pltpu.sync_copy(gbuf, acc.at[dstv], add=True)
        return carry
    lax.fori_loop(0, _NCHUNK, chunk, 0)

    plsc.subcore_barrier()
    pltpu.sync_copy(acc.at[pl.ds(s * _RPT, _RPT)],
                    out_hbm.at[c, pl.ds(s * _RPT, _RPT)])


def _sc_call(new_node, h, src, dst):
    mesh = plsc.VectorSubcoreMesh(core_axis_name="c", subcore_axis_name="s",
                                  num_cores=_NC, num_subcores=_NS)
    fn = functools.partial(
        pl.kernel,
        out_type=jax.ShapeDtypeStruct((_NC, _NPAD, DIM), jnp.float32),
        mesh=mesh,
        scratch_types=[
            pltpu.VMEM_SHARED((_NPAD, DIM), jnp.float32),
            pltpu.VMEM((_CH,), jnp.int32),
            pltpu.VMEM((_CH,), jnp.int32),
            pltpu.VMEM((_CH, DIM), jnp.float32),
            pltpu.VMEM((_CH, DIM), jnp.float32),
            pltpu.VMEM((_ZROWS, DIM), jnp.float32),
            pltpu.SemaphoreType.DMA,
        ],
    )(_sc_body)
    return fn(new_node, h, src, dst)


def _upd_body_next(p_ref, node_ref, w2_ref, b2_ref, w3_ref, b3_ref, w1n_ref,
                   nodeo_ref, newo_ref):
    cf = p_ref[0] + p_ref[1]
    t = _sp05(jnp.dot(cf, w2_ref[...], preferred_element_type=jnp.float32)
              + b2_ref[...])
    nn = (node_ref[...]
          + jnp.dot(t, w3_ref[...], preferred_element_type=jnp.float32)
          + b3_ref[...])
    nodeo_ref[...] = nn
    newo_ref[...] = jnp.dot(nn, w1n_ref[...], preferred_element_type=jnp.float32)


def _upd_body_last(p_ref, node_ref, w2_ref, b2_ref, w3_ref, b3_ref,
                   nodeo_ref):
    cf = p_ref[0] + p_ref[1]
    t = _sp05(jnp.dot(cf, w2_ref[...], preferred_element_type=jnp.float32)
              + b2_ref[...])
    nodeo_ref[...] = (node_ref[...]
                      + jnp.dot(t, w3_ref[...], preferred_element_type=jnp.float32)
                      + b3_ref[...])


def _upd_call(partials, node, w2, b2, w3, b3, w1next):
    mat = pl.BlockSpec((DIM, DIM), lambda i: (0, 0))
    vec = pl.BlockSpec((1, DIM), lambda i: (0, 0))
    row = pl.BlockSpec((1000, DIM), lambda i: (i, 0))
    in_specs = [pl.BlockSpec((_NC, 1000, DIM), lambda i: (0, i, 0)),
                row, mat, vec, mat, vec]
    args = [partials, node, w2, b2.reshape(1, DIM), w3, b3.reshape(1, DIM)]
    if w1next is not None:
        return pl.pallas_call(
            _upd_body_next,
            grid=(10,),
            in_specs=in_specs + [mat],
            out_specs=[row, row],
            out_shape=[jax.ShapeDtypeStruct((N, DIM), jnp.float32),
                       jax.ShapeDtypeStruct((N, DIM), jnp.float32)],
        )(*args, w1next)
    out = pl.pallas_call(
        _upd_body_last,
        grid=(10,),
        in_specs=in_specs,
        out_specs=row,
        out_shape=jax.ShapeDtypeStruct((N, DIM), jnp.float32),
    )(*args)
    return out, None


def _readout_body(node_ref, gid_ref, wd1_ref, bd1_ref, wd2_ref, bd2_ref,
                  preds_ref, sums, cnts):
    i = pl.program_id(0)

    @pl.when(i == 0)
    def _():
        sums[...] = jnp.zeros_like(sums)
        cnts[...] = jnp.zeros_like(cnts)

    atom = (jnp.dot(node_ref[...], wd1_ref[...], preferred_element_type=jnp.float32)
            + bd1_ref[...])
    res = _sp1(atom) - _LOG2
    g = gid_ref[0, 0, :]
    oh = (g[:, None] == lax.broadcasted_iota(jnp.int32, (1000, NG), 1)
          ).astype(jnp.float32)
    sums[...] += lax.dot_general(oh, res, (((0,), (0,)), ((), ())),
                                 preferred_element_type=jnp.float32)
    cnts[...] += lax.dot_general(oh, jnp.ones((1000, NG), jnp.float32),
                                 (((0,), (0,)), ((), ())),
                                 preferred_element_type=jnp.float32)

    @pl.when(i == 9)
    def _():
        mean = sums[...] / jnp.maximum(cnts[...], 1.0)
        preds_ref[...] = (jnp.dot(mean, wd2_ref[...],
                                  preferred_element_type=jnp.float32)
                          + bd2_ref[...])


def _readout_call(node, graph_ids, wd1, bd1, wd2, bd2):
    gid_r = graph_ids.reshape(10, 1, 1000)
    return pl.pallas_call(
        _readout_body,
        grid=(10,),
        in_specs=[
            pl.BlockSpec((1000, DIM), lambda i: (i, 0)),
            pl.BlockSpec((1, 1, 1000), lambda i: (i, 0, 0)),
            pl.BlockSpec((DIM, NG), lambda i: (0, 0)),
            pl.BlockSpec((1, NG), lambda i: (0, 0)),
            pl.BlockSpec((NG, 1), lambda i: (0, 0)),
            pl.BlockSpec((1, 1), lambda i: (0, 0)),
        ],
        out_specs=pl.BlockSpec((NG, 1), lambda i: (0, 0)),
        out_shape=jax.ShapeDtypeStruct((NG, 1), jnp.float32),
        scratch_shapes=[pltpu.VMEM((NG, NG), jnp.float32),
                        pltpu.VMEM((NG, NG), jnp.float32)],
    )(node, gid_r, wd1, bd1.reshape(1, NG), wd2, bd2.reshape(1, 1))


def kernel(nodes, edge_index, distance, graph_ids, emb, W1s, Wc1s, bc1s,
           Wc2s, bc2s, W2s, b2s, W3s, b3s, Wd1, bd1, Wd2, bd2):
    nodes = nodes.astype(jnp.int32)
    src = edge_index[0].astype(jnp.int32)
    dst = edge_index[1].astype(jnp.int32)
    graph_ids = graph_ids.astype(jnp.int32)

    node, new = _embed_call(nodes, emb, W1s[0])
    for i in range(NCONV):
        h = _h_call(distance, Wc1s[i], bc1s[i], Wc2s[i], bc2s[i])
        partials = _sc_call(new, h, src, dst)
        w1next = W1s[i + 1] if i + 1 < NCONV else None
        node, new = _upd_call(partials, node, W2s[i], b2s[i], W3s[i], b3s[i],
                              w1next)
    return _readout_call(node, graph_ids, Wd1, bd1, Wd2, bd2)
